# baseline (speedup 1.0000x reference)
import jax
import jax.numpy as jnp
from jax import lax
from jax.experimental import pallas as pl
from jax.experimental.pallas import tpu as pltpu
from jax.experimental.pallas import tpu_sc as plsc

NC, NS = 2, 16
NW = NC * NS
CP = 16
LANES = 16
NBUF = 4
PREF = 2
UNROLL = 8
BS = 512


def _sc_body(tok, pos, out, pb0, pb1, tb0, tb1, tb2, tb3,
             ps0, ps1, ls0, ls1, ls2, ls3, ss0, ss1, ss2, ss3):
    B, S, E = tok.shape
    b_own = B - 1
    spw = S // NW
    wid = lax.axis_index("s") * NC + lax.axis_index("c")
    base = wid * spw

    nchunks = spw // CP
    nvec = CP * E // LANES
    ecols = E // LANES

    tbufs = (tb0, tb1, tb2, tb3)
    lsems = (ls0, ls1, ls2, ls3)
    ssems = (ss0, ss1, ss2, ss3)
    pbufs = (pb0, pb1)
    psems = (ps0, ps1)

    load_d = [None] * NBUF
    store_d = [None] * NBUF
    pos_d = [None, None]

    pos_d[0] = pltpu.async_copy(pos.at[pl.ds(base, CP)], pbufs[0], psems[0])
    for up in range(min(PREF, nchunks)):
        load_d[up % NBUF] = pltpu.async_copy(
            tok.at[b_own, pl.ds(base + up * CP, CP)],
            tbufs[up % NBUF], lsems[up % NBUF])

    for u in range(nchunks):
        chunk = u
        s0 = base + chunk * CP
        pos_d[chunk % 2].wait()
        pos_d[chunk % 2] = None
        if chunk + 1 < nchunks:
            nxt = (chunk + 1) % 2
            pos_d[nxt] = pltpu.async_copy(
                pos.at[pl.ds(base + (chunk + 1) * CP, CP)],
                pbufs[nxt], psems[nxt])
        up = u + PREF
        if up < nchunks:
            slot = up % NBUF
            if store_d[slot] is not None:
                store_d[slot].wait()
                store_d[slot] = None
            load_d[slot] = pltpu.async_copy(
                tok.at[b_own, pl.ds(base + up * CP, CP)],
                tbufs[slot], lsems[slot])

        cur = u % NBUF
        load_d[cur].wait()
        load_d[cur] = None
        tb = tbufs[cur]
        pb = pbufs[chunk % 2]

        @plsc.parallel_loop(0, nvec, 1, unroll=UNROLL)
        def body(i):
            r = i // ecols
            sl = pl.ds((i % ecols) * LANES, LANES)
            plsc.addupdate(tb.at[r, sl], pb[r, sl])

        store_d[cur] = pltpu.async_copy(
            tb, out.at[pl.ds(chunk * CP, CP)], ssems[cur])

    for slot in range(NBUF):
        if store_d[slot] is not None:
            store_d[slot].wait()


def _sc_part(token_embedding, pos_table):
    B, S, E = token_embedding.shape
    f = pl.kernel(
        _sc_body,
        out_type=jax.ShapeDtypeStruct((S, E), token_embedding.dtype),
        mesh=plsc.VectorSubcoreMesh(core_axis_name="c", subcore_axis_name="s"),
        scratch_types=[
            pltpu.VMEM((CP, E), jnp.float32),
            pltpu.VMEM((CP, E), jnp.float32),
            pltpu.VMEM((CP, E), jnp.float32),
            pltpu.VMEM((CP, E), jnp.float32),
            pltpu.VMEM((CP, E), jnp.float32),
            pltpu.VMEM((CP, E), jnp.float32),
            pltpu.SemaphoreType.DMA,
            pltpu.SemaphoreType.DMA,
            pltpu.SemaphoreType.DMA,
            pltpu.SemaphoreType.DMA,
            pltpu.SemaphoreType.DMA,
            pltpu.SemaphoreType.DMA,
            pltpu.SemaphoreType.DMA,
            pltpu.SemaphoreType.DMA,
            pltpu.SemaphoreType.DMA,
            pltpu.SemaphoreType.DMA,
        ],
    )
    return f(token_embedding, pos_table)


def _tc_body(tok_ref, pos_ref, out_ref):
    out_ref[...] = tok_ref[...] + pos_ref[...][None, :, :]


def _tc_part(token_embedding, pos_table, nb):
    B, S, E = token_embedding.shape
    grid = (S // BS, nb)
    return pl.pallas_call(
        _tc_body,
        grid=grid,
        in_specs=[
            pl.BlockSpec((1, BS, E), lambda i, b: (b, i, 0)),
            pl.BlockSpec((BS, E), lambda i, b: (i, 0)),
        ],
        out_specs=pl.BlockSpec((1, BS, E), lambda i, b: (b, i, 0)),
        out_shape=jax.ShapeDtypeStruct((nb, S, E), token_embedding.dtype),
    )(token_embedding, pos_table)


def kernel(token_embedding, pos_table):
    B, S, E = token_embedding.shape
    sc_out = _sc_part(token_embedding, pos_table)
    tc_out = _tc_part(token_embedding, pos_table, B - 1)
    return jnp.concatenate([tc_out, sc_out[None]], axis=0)


# speedup vs baseline: 1.0624x; 1.0624x over previous
import jax
import jax.numpy as jnp
from jax import lax
from jax.experimental import pallas as pl
from jax.experimental.pallas import tpu as pltpu
from jax.experimental.pallas import tpu_sc as plsc

NC, NS = 2, 16
NW = NC * NS
CP = 16
LANES = 16
NBUF = 4
PREF = 2
UNROLL = 8
BS = 512


def _sc_body(tok, pos, out, pb0, pb1, tb0, tb1, tb2, tb3,
             ps0, ps1, ls0, ls1, ls2, ls3, ss0, ss1, ss2, ss3):
    B, S, E = tok.shape
    b_own = B - 1
    spw = S // NW
    wid = lax.axis_index("s") * NC + lax.axis_index("c")
    base = wid * spw

    nchunks = spw // CP
    nvec = CP * E // LANES
    ecols = E // LANES

    tbufs = (tb0, tb1, tb2, tb3)
    lsems = (ls0, ls1, ls2, ls3)
    ssems = (ss0, ss1, ss2, ss3)
    pbufs = (pb0, pb1)
    psems = (ps0, ps1)

    load_d = [None] * NBUF
    store_d = [None] * NBUF
    pos_d = [None, None]

    pos_d[0] = pltpu.async_copy(pos.at[pl.ds(base, CP)], pbufs[0], psems[0])
    for up in range(min(PREF, nchunks)):
        load_d[up % NBUF] = pltpu.async_copy(
            tok.at[b_own, pl.ds(base + up * CP, CP)],
            tbufs[up % NBUF], lsems[up % NBUF])

    for u in range(nchunks):
        chunk = u
        s0 = base + chunk * CP
        pos_d[chunk % 2].wait()
        pos_d[chunk % 2] = None
        if chunk + 1 < nchunks:
            nxt = (chunk + 1) % 2
            pos_d[nxt] = pltpu.async_copy(
                pos.at[pl.ds(base + (chunk + 1) * CP, CP)],
                pbufs[nxt], psems[nxt])
        up = u + PREF
        if up < nchunks:
            slot = up % NBUF
            if store_d[slot] is not None:
                store_d[slot].wait()
                store_d[slot] = None
            load_d[slot] = pltpu.async_copy(
                tok.at[b_own, pl.ds(base + up * CP, CP)],
                tbufs[slot], lsems[slot])

        cur = u % NBUF
        load_d[cur].wait()
        load_d[cur] = None
        tb = tbufs[cur]
        pb = pbufs[chunk % 2]

        @plsc.parallel_loop(0, nvec, 1, unroll=UNROLL)
        def body(i):
            r = i // ecols
            sl = pl.ds((i % ecols) * LANES, LANES)
            plsc.addupdate(tb.at[r, sl], pb[r, sl])

        store_d[cur] = pltpu.async_copy(
            tb, out.at[pl.ds(s0, CP)], ssems[cur])

    for slot in range(NBUF):
        if store_d[slot] is not None:
            store_d[slot].wait()


def _sc_part(token_embedding, pos_table):
    B, S, E = token_embedding.shape
    f = pl.kernel(
        _sc_body,
        out_type=jax.ShapeDtypeStruct((S, E), token_embedding.dtype),
        mesh=plsc.VectorSubcoreMesh(core_axis_name="c", subcore_axis_name="s"),
        scratch_types=[
            pltpu.VMEM((CP, E), jnp.float32),
            pltpu.VMEM((CP, E), jnp.float32),
            pltpu.VMEM((CP, E), jnp.float32),
            pltpu.VMEM((CP, E), jnp.float32),
            pltpu.VMEM((CP, E), jnp.float32),
            pltpu.VMEM((CP, E), jnp.float32),
            pltpu.SemaphoreType.DMA,
            pltpu.SemaphoreType.DMA,
            pltpu.SemaphoreType.DMA,
            pltpu.SemaphoreType.DMA,
            pltpu.SemaphoreType.DMA,
            pltpu.SemaphoreType.DMA,
            pltpu.SemaphoreType.DMA,
            pltpu.SemaphoreType.DMA,
            pltpu.SemaphoreType.DMA,
            pltpu.SemaphoreType.DMA,
        ],
    )
    return f(token_embedding, pos_table)


def _tc_body(tok_ref, pos_ref, out_ref):
    out_ref[...] = tok_ref[...] + pos_ref[...][None, :, :]


def _tc_part(token_embedding, pos_table, nb):
    B, S, E = token_embedding.shape
    grid = (S // BS, nb)
    return pl.pallas_call(
        _tc_body,
        grid=grid,
        in_specs=[
            pl.BlockSpec((1, BS, E), lambda i, b: (b, i, 0)),
            pl.BlockSpec((BS, E), lambda i, b: (i, 0)),
        ],
        out_specs=pl.BlockSpec((1, BS, E), lambda i, b: (b, i, 0)),
        out_shape=jax.ShapeDtypeStruct((nb, S, E), token_embedding.dtype),
    )(token_embedding, pos_table)


def kernel(token_embedding, pos_table):
    B, S, E = token_embedding.shape
    sc_out = _sc_part(token_embedding, pos_table)
    tc_out = _tc_part(token_embedding, pos_table, B - 1)
    return jnp.concatenate([tc_out, sc_out[None]], axis=0)


# speedup vs baseline: 1.4853x; 1.3980x over previous
import jax
import jax.numpy as jnp
from jax import lax
from jax.experimental import pallas as pl
from jax.experimental.pallas import tpu as pltpu
from jax.experimental.pallas import tpu_sc as plsc

NC, NS = 2, 16
NW = NC * NS
CP = 16
LANES = 16
NBUF = 4
PREF = 3
UNROLL = 8


def _sc_body(tok, pos, out, pb0, pb1, tb0, tb1, tb2, tb3,
             ps0, ps1, ls0, ls1, ls2, ls3, ss0, ss1, ss2, ss3):
    B, S, E = tok.shape
    spw = S // NW
    wid = lax.axis_index("s") * NC + lax.axis_index("c")
    base = wid * spw

    nchunks = spw // CP
    nunits = nchunks * B
    nvec = CP * E // LANES
    ecols = E // LANES

    tbufs = (tb0, tb1, tb2, tb3)
    lsems = (ls0, ls1, ls2, ls3)
    ssems = (ss0, ss1, ss2, ss3)
    pbufs = (pb0, pb1)
    psems = (ps0, ps1)

    load_d = [None] * NBUF
    store_d = [None] * NBUF
    pos_d = [None, None]

    def unit_pos(u):
        chunk, b = u // B, u % B
        return chunk, b, base + chunk * CP

    pos_d[0] = pltpu.async_copy(pos.at[pl.ds(base, CP)], pbufs[0], psems[0])
    for up in range(min(PREF, nunits)):
        _, b, s0 = unit_pos(up)
        load_d[up % NBUF] = pltpu.async_copy(
            tok.at[b, pl.ds(s0, CP)], tbufs[up % NBUF], lsems[up % NBUF])

    for u in range(nunits):
        chunk, b, s0 = unit_pos(u)
        if b == 0:
            pos_d[chunk % 2].wait()
            pos_d[chunk % 2] = None
            if chunk + 1 < nchunks:
                nxt = (chunk + 1) % 2
                pos_d[nxt] = pltpu.async_copy(
                    pos.at[pl.ds(base + (chunk + 1) * CP, CP)],
                    pbufs[nxt], psems[nxt])
        up = u + PREF
        if up < nunits:
            slot = up % NBUF
            if store_d[slot] is not None:
                store_d[slot].wait()
                store_d[slot] = None
            _, ub, us0 = unit_pos(up)
            load_d[slot] = pltpu.async_copy(
                tok.at[ub, pl.ds(us0, CP)], tbufs[slot], lsems[slot])

        cur = u % NBUF
        load_d[cur].wait()
        load_d[cur] = None
        tb = tbufs[cur]
        pb = pbufs[chunk % 2]

        @plsc.parallel_loop(0, nvec, 1, unroll=UNROLL)
        def body(i):
            r = i // ecols
            sl = pl.ds((i % ecols) * LANES, LANES)
            plsc.addupdate(tb.at[r, sl], pb[r, sl])

        store_d[cur] = pltpu.async_copy(
            tb, out.at[b, pl.ds(s0, CP)], ssems[cur])

    for slot in range(NBUF):
        if store_d[slot] is not None:
            store_d[slot].wait()


def kernel(token_embedding, pos_table):
    B, S, E = token_embedding.shape
    f = pl.kernel(
        _sc_body,
        out_type=jax.ShapeDtypeStruct((B, S, E), token_embedding.dtype),
        mesh=plsc.VectorSubcoreMesh(core_axis_name="c", subcore_axis_name="s"),
        scratch_types=[
            pltpu.VMEM((CP, E), jnp.float32),
            pltpu.VMEM((CP, E), jnp.float32),
            pltpu.VMEM((CP, E), jnp.float32),
            pltpu.VMEM((CP, E), jnp.float32),
            pltpu.VMEM((CP, E), jnp.float32),
            pltpu.VMEM((CP, E), jnp.float32),
            pltpu.SemaphoreType.DMA,
            pltpu.SemaphoreType.DMA,
            pltpu.SemaphoreType.DMA,
            pltpu.SemaphoreType.DMA,
            pltpu.SemaphoreType.DMA,
            pltpu.SemaphoreType.DMA,
            pltpu.SemaphoreType.DMA,
            pltpu.SemaphoreType.DMA,
            pltpu.SemaphoreType.DMA,
            pltpu.SemaphoreType.DMA,
        ],
    )
    return f(token_embedding, pos_table)


# speedup vs baseline: 1.6216x; 1.0917x over previous
import jax
import jax.numpy as jnp
from jax import lax
from jax.experimental import pallas as pl
from jax.experimental.pallas import tpu as pltpu
from jax.experimental.pallas import tpu_sc as plsc

NC, NS = 2, 16
NW = NC * NS
CP = 16
LANES = 16
NBUF = 4
PREF = 2
UNROLL = 8


def _sc_body(tok, pos, out, pb0, pb1, tb0, tb1, tb2, tb3,
             ps0, ps1, ls0, ls1, ls2, ls3, ss0, ss1, ss2, ss3):
    B, S, E = tok.shape
    spw = S // NW
    wid = lax.axis_index("s") * NC + lax.axis_index("c")
    base = wid * spw

    nchunks = spw // CP
    nunits = nchunks * B
    nvec = CP * E // LANES
    ecols = E // LANES

    tbufs = (tb0, tb1, tb2, tb3)
    lsems = (ls0, ls1, ls2, ls3)
    ssems = (ss0, ss1, ss2, ss3)
    pbufs = (pb0, pb1)
    psems = (ps0, ps1)

    load_d = [None] * NBUF
    store_d = [None] * NBUF
    pos_d = [None, None]

    def unit_pos(u):
        chunk, b = u // B, u % B
        return chunk, b, base + chunk * CP

    pos_d[0] = pltpu.async_copy(pos.at[pl.ds(base, CP)], pbufs[0], psems[0])
    for up in range(min(PREF, nunits)):
        _, b, s0 = unit_pos(up)
        load_d[up % NBUF] = pltpu.async_copy(
            tok.at[b, pl.ds(s0, CP)], tbufs[up % NBUF], lsems[up % NBUF])

    for u in range(nunits):
        chunk, b, s0 = unit_pos(u)
        if b == 0:
            pos_d[chunk % 2].wait()
            pos_d[chunk % 2] = None
            if chunk + 1 < nchunks:
                nxt = (chunk + 1) % 2
                pos_d[nxt] = pltpu.async_copy(
                    pos.at[pl.ds(base + (chunk + 1) * CP, CP)],
                    pbufs[nxt], psems[nxt])
        up = u + PREF
        if up < nunits:
            slot = up % NBUF
            if store_d[slot] is not None:
                store_d[slot].wait()
                store_d[slot] = None
            _, ub, us0 = unit_pos(up)
            load_d[slot] = pltpu.async_copy(
                tok.at[ub, pl.ds(us0, CP)], tbufs[slot], lsems[slot])

        cur = u % NBUF
        load_d[cur].wait()
        load_d[cur] = None
        tb = tbufs[cur]
        pb = pbufs[chunk % 2]

        @plsc.parallel_loop(0, nvec, 1, unroll=UNROLL)
        def body(i):
            r = i // ecols
            sl = pl.ds((i % ecols) * LANES, LANES)
            plsc.addupdate(tb.at[r, sl], pb[r, sl])

        store_d[cur] = pltpu.async_copy(
            tb, out.at[b, pl.ds(s0, CP)], ssems[cur])

    for slot in range(NBUF):
        if store_d[slot] is not None:
            store_d[slot].wait()


def kernel(token_embedding, pos_table):
    B, S, E = token_embedding.shape
    f = pl.kernel(
        _sc_body,
        out_type=jax.ShapeDtypeStruct((B, S, E), token_embedding.dtype),
        mesh=plsc.VectorSubcoreMesh(core_axis_name="c", subcore_axis_name="s"),
        scratch_types=[
            pltpu.VMEM((CP, E), jnp.float32),
            pltpu.VMEM((CP, E), jnp.float32),
            pltpu.VMEM((CP, E), jnp.float32),
            pltpu.VMEM((CP, E), jnp.float32),
            pltpu.VMEM((CP, E), jnp.float32),
            pltpu.VMEM((CP, E), jnp.float32),
            pltpu.SemaphoreType.DMA,
            pltpu.SemaphoreType.DMA,
            pltpu.SemaphoreType.DMA,
            pltpu.SemaphoreType.DMA,
            pltpu.SemaphoreType.DMA,
            pltpu.SemaphoreType.DMA,
            pltpu.SemaphoreType.DMA,
            pltpu.SemaphoreType.DMA,
            pltpu.SemaphoreType.DMA,
            pltpu.SemaphoreType.DMA,
        ],
    )
    return f(token_embedding, pos_table)
